# baseline (speedup 1.0000x reference)
import jax
import jax.numpy as jnp
from jax.experimental import pallas as pl

_STEPS_PAD = 1024
_BB = 128
_CB = 8192


def _scale_kernel(ts_ref, a_ref, s_ref, x_ref, n_ref, o_ref):
    t = ts_ref[0, 0, :]
    iota = jax.lax.broadcasted_iota(jnp.int32, (_BB, _STEPS_PAD), 1)
    onehot = iota == t[:, None]
    ca = jnp.sum(jnp.where(onehot, a_ref[0, :][None, :], 0.0), axis=1)
    cs = jnp.sum(jnp.where(onehot, s_ref[0, :][None, :], 0.0), axis=1)
    o_ref[:, :] = ca[:, None] * x_ref[:, :] + cs[:, None] * n_ref[:, :]


def kernel(x0, noise, timesteps, sqrt_alphas_cumprod, sqrt_one_minus_alphas_cumprod):
    B = x0.shape[0]
    cols = x0.size // B
    x = x0.reshape(B, cols)
    n = noise.reshape(B, cols)
    nb = B // _BB
    ts3 = timesteps.reshape(nb, 1, _BB)
    steps = sqrt_alphas_cumprod.shape[0]
    a_p = jnp.zeros((1, _STEPS_PAD), x0.dtype).at[0, :steps].set(sqrt_alphas_cumprod)
    s_p = jnp.zeros((1, _STEPS_PAD), x0.dtype).at[0, :steps].set(
        sqrt_one_minus_alphas_cumprod)

    out = pl.pallas_call(
        _scale_kernel,
        grid=(nb, cols // _CB),
        in_specs=[
            pl.BlockSpec((1, 1, _BB), lambda i, j: (i, 0, 0)),
            pl.BlockSpec((1, _STEPS_PAD), lambda i, j: (0, 0)),
            pl.BlockSpec((1, _STEPS_PAD), lambda i, j: (0, 0)),
            pl.BlockSpec((_BB, _CB), lambda i, j: (i, j)),
            pl.BlockSpec((_BB, _CB), lambda i, j: (i, j)),
        ],
        out_specs=pl.BlockSpec((_BB, _CB), lambda i, j: (i, j)),
        out_shape=jax.ShapeDtypeStruct((B, cols), x0.dtype),
    )(ts3, a_p, s_p, x, n)
    return out.reshape(x0.shape)
